# baseline (speedup 1.0000x reference)
import functools

import jax
import jax.numpy as jnp
from jax import lax
from jax.experimental import pallas as pl
from jax.experimental.pallas import tpu as pltpu
from jax.experimental.pallas import tpu_sc as plsc

_DP = 128


def _make_remap_kernel(N, n_workers, chunk):
    per_w = N // n_workers
    n_chunks = per_w // chunk
    assert n_chunks % 2 == 0 and n_chunks >= 4
    mesh = plsc.VectorSubcoreMesh(core_axis_name="c", subcore_axis_name="s")

    @functools.partial(
        pl.kernel,
        mesh=mesh,
        compiler_params=pltpu.CompilerParams(use_tc_tiling_on_sc=False),
        out_type=jax.ShapeDtypeStruct((N,), jnp.int32),
        scratch_types=[
            pltpu.VMEM((2 * chunk,), jnp.int32),
            pltpu.VMEM((2 * chunk,), jnp.int32),
            pltpu.SemaphoreType.DMA,
            pltpu.SemaphoreType.DMA,
            pltpu.SemaphoreType.DMA,
            pltpu.SemaphoreType.DMA,
        ],
    )
    def k(words_hbm, w2w2_hbm, out_hbm, idx_v, remap_v, rs0, rs1, os0, os1):
        wid = lax.axis_index("s") * 2 + lax.axis_index("c")
        base = wid * per_w
        rsem = (rs0, rs1)
        osem = (os0, os1)

        def isl(p):
            return pl.ds(p * chunk, chunk)

        for p in (0, 1):
            pltpu.sync_copy(words_hbm.at[pl.ds(base + p * chunk, chunk)],
                            idx_v.at[isl(p)])
            pltpu.async_copy(w2w2_hbm.at[idx_v.at[isl(p)]],
                             remap_v.at[isl(p)], rsem[p])

        def pair(i, carry):
            for p in (0, 1):
                g = i * 2 + p
                off = base + g * chunk

                @pl.when(g >= 2)
                def _drain_and_regather():
                    pltpu.make_async_copy(
                        remap_v.at[isl(p)],
                        out_hbm.at[pl.ds(off - 2 * chunk, chunk)],
                        osem[p],
                    ).wait()
                    pltpu.async_copy(w2w2_hbm.at[idx_v.at[isl(p)]],
                                     remap_v.at[isl(p)], rsem[p])

                pltpu.make_async_copy(
                    w2w2_hbm.at[idx_v.at[isl(p)]],
                    remap_v.at[isl(p)], rsem[p]
                ).wait()
                pltpu.async_copy(
                    remap_v.at[isl(p)], out_hbm.at[pl.ds(off, chunk)],
                    osem[p],
                )

                @pl.when(g + 2 < n_chunks)
                def _prefetch_idx():
                    pltpu.sync_copy(
                        words_hbm.at[pl.ds(off + 2 * chunk, chunk)],
                        idx_v.at[isl(p)],
                    )

            return carry

        lax.fori_loop(0, n_chunks // 2, pair, 0)

        for p in (0, 1):
            g = n_chunks - 2 + p
            pltpu.make_async_copy(
                remap_v.at[isl(p)],
                out_hbm.at[pl.ds(base + g * chunk, chunk)],
                osem[p],
            ).wait()

    return k


def _make_row_kernel(N, D, n_workers, chunk):
    per_w = N // n_workers
    n_chunks = per_w // chunk
    assert n_chunks % 2 == 0 and n_chunks >= 4
    mesh = plsc.VectorSubcoreMesh(core_axis_name="c", subcore_axis_name="s")

    @functools.partial(
        pl.kernel,
        mesh=mesh,
        compiler_params=pltpu.CompilerParams(use_tc_tiling_on_sc=False),
        out_type=jax.ShapeDtypeStruct((N, _DP), jnp.float32),
        scratch_types=[
            pltpu.VMEM((2 * chunk,), jnp.int32),
            pltpu.VMEM((2, chunk, D), jnp.float32),
            pltpu.SemaphoreType.DMA,
            pltpu.SemaphoreType.DMA,
            pltpu.SemaphoreType.DMA,
            pltpu.SemaphoreType.DMA,
        ],
    )
    def k(remap_hbm, emb_hbm, out_hbm, remap_v, rows_v, gs0, gs1, os0, os1):
        wid = lax.axis_index("s") * 2 + lax.axis_index("c")
        base = wid * per_w
        gsem = (gs0, gs1)
        osem = (os0, os1)

        def isl(p):
            return pl.ds(p * chunk, chunk)

        for p in (0, 1):
            pltpu.sync_copy(remap_hbm.at[pl.ds(base + p * chunk, chunk)],
                            remap_v.at[isl(p)])
        pltpu.async_copy(
            emb_hbm.at[remap_v.at[isl(0)]], rows_v.at[0], gsem[0]
        )

        def pair(i, carry):
            for p in (0, 1):
                g = i * 2 + p
                off = base + g * chunk
                q = 1 - p

                @pl.when(g >= 1)
                def _drain_prev_write():
                    pltpu.make_async_copy(
                        rows_v.at[q],
                        out_hbm.at[pl.ds(off - chunk, chunk), pl.ds(0, D)],
                        osem[q],
                    ).wait()

                @pl.when(g + 1 < n_chunks)
                def _start_next_gather():
                    pltpu.async_copy(
                        emb_hbm.at[remap_v.at[isl(q)]], rows_v.at[q], gsem[q]
                    )

                pltpu.make_async_copy(
                    emb_hbm.at[remap_v.at[isl(p)]], rows_v.at[p], gsem[p]
                ).wait()
                pltpu.async_copy(
                    rows_v.at[p],
                    out_hbm.at[pl.ds(off, chunk), pl.ds(0, D)],
                    osem[p],
                )

                @pl.when(g + 2 < n_chunks)
                def _prefetch_idx():
                    pltpu.sync_copy(
                        remap_hbm.at[pl.ds(off + 2 * chunk, chunk)],
                        remap_v.at[isl(p)],
                    )

            return carry

        lax.fori_loop(0, n_chunks // 2, pair, 0)

        pltpu.make_async_copy(
            rows_v.at[1],
            out_hbm.at[pl.ds(base + (n_chunks - 1) * chunk, chunk),
                       pl.ds(0, D)],
            osem[1],
        ).wait()

    return k


def kernel(words, words_to_words, embedding_weight):
    B, L = words.shape
    V, D = embedding_weight.shape
    N = B * L
    words_flat = words.reshape(N).astype(jnp.int32)
    w2w2 = words_to_words * 2
    emb_padded = jnp.pad(embedding_weight, ((0, 0), (0, _DP - D)))
    emb_rows = emb_padded.reshape(V * _DP // D, D)
    ka = _make_remap_kernel(N, n_workers=32, chunk=1600)
    kb = _make_row_kernel(N, D, n_workers=32, chunk=800)
    remap2 = ka(words_flat, w2w2)
    out = kb(remap2, emb_rows)
    return out[:, :D].reshape(B, L, D)
